# baseline (speedup 1.0000x reference)
import functools

import jax
import jax.numpy as jnp
from jax import lax
from jax.experimental import pallas as pl
from jax.experimental.pallas import tpu as pltpu
from jax.experimental.pallas import tpu_sc as plsc

B, L = 4096, 200
V, D = 1000, 64
VP = 1024
T = 100

NC, NS = 2, 16
NW = NC * NS
ROWS_PER_W = B // NW
CH = 32
NCH = ROWS_PER_W // CH

UNROLL = 8


def _hist_body(sent_hbm, counts_hbm, sent_v, counts_v):
    wid = lax.axis_index("s") * NC + lax.axis_index("c")
    lanes = lax.iota(jnp.int32, 16)
    zeros16 = jnp.zeros((16,), jnp.float32)
    ones16 = jnp.ones((16,), jnp.float32)

    def zrow(r, carry):
        def zcol(cb, c2):
            for j in range(UNROLL):
                counts_v[r, pl.ds((cb * UNROLL + j) * 16, 16)] = zeros16
            return c2

        lax.fori_loop(0, VP // (16 * UNROLL), zcol, None)
        return carry

    lax.fori_loop(0, CH, zrow, None)

    def chunk_body(c, _):
        base = wid * ROWS_PER_W + c * CH
        pltpu.sync_copy(sent_hbm.at[pl.ds(base, CH)], sent_v)

        def grp(g, carry):
            row = g * 16 + lanes

            def lbody(lb, c2):
                for j in range(UNROLL):
                    l = jnp.zeros((16,), jnp.int32) + (lb * UNROLL + j)
                    col = plsc.load_gather(sent_v, [row, l])
                    plsc.addupdate_scatter(counts_v, [row, col], ones16)
                return c2

            lax.fori_loop(0, L // UNROLL, lbody, None)
            return carry

        lax.fori_loop(0, CH // 16, grp, None)
        pltpu.sync_copy(counts_v, counts_hbm.at[pl.ds(base, CH)])

        def rgrp(g, carry):
            row = g * 16 + lanes

            def lbody(lb, c2):
                for j in range(UNROLL):
                    l = jnp.zeros((16,), jnp.int32) + (lb * UNROLL + j)
                    col = plsc.load_gather(sent_v, [row, l])
                    plsc.store_scatter(counts_v, [row, col], zeros16)
                return c2

            lax.fori_loop(0, L // UNROLL, lbody, None)
            return carry

        lax.fori_loop(0, CH // 16, rgrp, None)
        return _

    lax.fori_loop(0, NCH, chunk_body, None)


@functools.cache
def _make_hist():
    mesh = plsc.VectorSubcoreMesh(core_axis_name="c", subcore_axis_name="s")
    return functools.partial(
        pl.kernel,
        mesh=mesh,
        out_type=jax.ShapeDtypeStruct((B, VP), jnp.float32),
        scratch_types=[
            pltpu.VMEM((CH, L), jnp.int32),
            pltpu.VMEM((CH, VP), jnp.float32),
        ],
        compiler_params=pltpu.CompilerParams(needs_layout_passes=False),
    )(_hist_body)


BB = 512


def _tc_body(counts_ref, table_ref, w_ref, b_ref, out_ref):
    bow = jnp.dot(counts_ref[...].astype(jnp.bfloat16),
                  table_ref[...].astype(jnp.bfloat16),
                  preferred_element_type=jnp.float32)
    sig = 1.0 / (1.0 + jnp.exp(-bow))
    tag = lax.dot_general(sig, w_ref[...], (((1,), (1,)), ((), ())),
                          preferred_element_type=jnp.float32)
    out_ref[...] = tag + b_ref[...]


def _tc_call(counts, table, w, b2d):
    return pl.pallas_call(
        _tc_body,
        grid=(B // BB,),
        in_specs=[
            pl.BlockSpec((BB, VP), lambda i: (i, 0)),
            pl.BlockSpec((VP, D), lambda i: (0, 0)),
            pl.BlockSpec((T, D), lambda i: (0, 0)),
            pl.BlockSpec((1, T), lambda i: (0, 0)),
        ],
        out_specs=pl.BlockSpec((BB, T), lambda i: (i, 0)),
        out_shape=jax.ShapeDtypeStruct((B, T), jnp.float32),
    )(counts, table, w, b2d)


def kernel(sentence, emb_table, W, b):
    counts = _make_hist()(sentence.astype(jnp.int32))
    table_p = jnp.pad(emb_table, ((0, VP - V), (0, 0)))
    return _tc_call(counts, table_p, W, b.reshape(1, T))


# speedup vs baseline: 1.0397x; 1.0397x over previous
import functools

import jax
import jax.numpy as jnp
from jax import lax
from jax.experimental import pallas as pl
from jax.experimental.pallas import tpu as pltpu
from jax.experimental.pallas import tpu_sc as plsc

B, L = 4096, 200
V, D = 1000, 64
VP = 1024
T = 100

NC, NS = 2, 16
NW = NC * NS
ROWS_PER_W = B // NW
CH = 32
NCH = ROWS_PER_W // CH

UNROLL = 8


def _hist_body(sent_hbm, counts_hbm, sent_v, counts_v):
    wid = lax.axis_index("s") * NC + lax.axis_index("c")
    lanes = lax.iota(jnp.int32, 16)
    zeros16 = jnp.zeros((16,), jnp.float32)
    ones16 = jnp.ones((16,), jnp.float32)

    def zbody(i, carry):
        for j in range(UNROLL):
            counts_v[pl.ds(i * 16 * UNROLL + j * 16, 16)] = zeros16
        return carry

    lax.fori_loop(0, CH * VP // (16 * UNROLL), zbody, None)

    def chunk_body(c, _):
        base = wid * ROWS_PER_W + c * CH
        pltpu.sync_copy(sent_hbm.at[pl.ds(base * L, CH * L)], sent_v)

        def grp(g, carry):
            row = g * 16 + lanes
            rowoff_s = row * L
            rowoff_c = row * VP

            def lbody(lb, c2):
                for j in range(UNROLL):
                    col = plsc.load_gather(sent_v, [rowoff_s + (lb * UNROLL + j)])
                    plsc.addupdate_scatter(counts_v, [rowoff_c + col], ones16)
                return c2

            lax.fori_loop(0, L // UNROLL, lbody, None)
            return carry

        lax.fori_loop(0, CH // 16, grp, None)
        pltpu.sync_copy(counts_v, counts_hbm.at[pl.ds(base * VP, CH * VP)])

        def rgrp(g, carry):
            row = g * 16 + lanes
            rowoff_s = row * L
            rowoff_c = row * VP

            def lbody(lb, c2):
                for j in range(UNROLL):
                    col = plsc.load_gather(sent_v, [rowoff_s + (lb * UNROLL + j)])
                    plsc.store_scatter(counts_v, [rowoff_c + col], zeros16)
                return c2

            lax.fori_loop(0, L // UNROLL, lbody, None)
            return carry

        lax.fori_loop(0, CH // 16, rgrp, None)
        return _

    lax.fori_loop(0, NCH, chunk_body, None)


@functools.cache
def _make_hist():
    mesh = plsc.VectorSubcoreMesh(core_axis_name="c", subcore_axis_name="s")
    return functools.partial(
        pl.kernel,
        mesh=mesh,
        out_type=jax.ShapeDtypeStruct((B * VP,), jnp.float32),
        scratch_types=[
            pltpu.VMEM((CH * L,), jnp.int32),
            pltpu.VMEM((CH * VP,), jnp.float32),
        ],
        compiler_params=pltpu.CompilerParams(needs_layout_passes=False),
    )(_hist_body)


BB = 512


def _tc_body(counts_ref, table_ref, w_ref, b_ref, out_ref):
    bow = jnp.dot(counts_ref[...].astype(jnp.bfloat16),
                  table_ref[...].astype(jnp.bfloat16),
                  preferred_element_type=jnp.float32)
    sig = 1.0 / (1.0 + jnp.exp(-bow))
    tag = lax.dot_general(sig, w_ref[...], (((1,), (1,)), ((), ())),
                          preferred_element_type=jnp.float32)
    out_ref[...] = tag + b_ref[...]


def _tc_call(counts, table, w, b2d):
    return pl.pallas_call(
        _tc_body,
        grid=(B // BB,),
        in_specs=[
            pl.BlockSpec((BB, VP), lambda i: (i, 0)),
            pl.BlockSpec((VP, D), lambda i: (0, 0)),
            pl.BlockSpec((T, D), lambda i: (0, 0)),
            pl.BlockSpec((1, T), lambda i: (0, 0)),
        ],
        out_specs=pl.BlockSpec((BB, T), lambda i: (i, 0)),
        out_shape=jax.ShapeDtypeStruct((B, T), jnp.float32),
    )(counts, table, w, b2d)


def kernel(sentence, emb_table, W, b):
    sent_flat = sentence.reshape(B * L).astype(jnp.int32)
    counts = _make_hist()(sent_flat).reshape(B, VP)
    table_p = jnp.pad(emb_table, ((0, VP - V), (0, 0)))
    return _tc_call(counts, table_p, W, b.reshape(1, T))


# speedup vs baseline: 1.2614x; 1.2131x over previous
import functools

import jax
import jax.numpy as jnp
from jax import lax
from jax.experimental import pallas as pl
from jax.experimental.pallas import tpu as pltpu
from jax.experimental.pallas import tpu_sc as plsc

B, L = 4096, 200
V, D = 1000, 64
VP = 1024
KS = VP // 128
T = 100

NC, NS = 2, 16
NW = NC * NS
ROWS_PER_W = B // NW
CH = 32
NCH = ROWS_PER_W // CH

UNROLL = 8


def _hist_body(sent_hbm, counts_hbm, sent_v, counts_v2, sem):
    wid = lax.axis_index("s") * NC + lax.axis_index("c")
    lanes = lax.iota(jnp.int32, 16)
    zeros16 = jnp.zeros((16,), jnp.float32)
    zeros_i = jnp.zeros((16,), jnp.int32)
    ones16 = jnp.ones((16,), jnp.float32)
    counts_3d = counts_v2.reshape(CH, KS, 128)

    def zbody(i, carry):
        for j in range(UNROLL):
            counts_v2[0, pl.ds(i * 16 * UNROLL + j * 16, 16)] = zeros16
        return carry

    lax.fori_loop(0, CH * VP // (16 * UNROLL), zbody, None)

    def chunk_body(c, _):
        base = wid * ROWS_PER_W + c * CH
        pltpu.sync_copy(sent_hbm.at[pl.ds(base * L, CH * L)], sent_v)

        def grp(g, carry):
            row = g * 16 + lanes
            rowoff_s = row * L
            rowoff_c = row * VP

            def lbody(lb, c2):
                for j in range(UNROLL):
                    col = plsc.load_gather(sent_v, [rowoff_s + (lb * UNROLL + j)])
                    plsc.addupdate_scatter(counts_v2, [zeros_i, rowoff_c + col],
                                           ones16)
                return c2

            lax.fori_loop(0, L // UNROLL, lbody, None)
            return carry

        lax.fori_loop(0, CH // 16, grp, None)

        for k in range(KS):
            pltpu.async_copy(counts_3d.at[:, k, :],
                             counts_hbm.at[k, pl.ds(base, CH)], sem)
        for k in range(KS):
            pltpu.make_async_copy(counts_3d.at[:, k, :],
                                  counts_hbm.at[k, pl.ds(base, CH)], sem).wait()

        def rgrp(g, carry):
            row = g * 16 + lanes
            rowoff_s = row * L
            rowoff_c = row * VP

            def lbody(lb, c2):
                for j in range(UNROLL):
                    col = plsc.load_gather(sent_v, [rowoff_s + (lb * UNROLL + j)])
                    plsc.store_scatter(counts_v2, [zeros_i, rowoff_c + col],
                                       zeros16)
                return c2

            lax.fori_loop(0, L // UNROLL, lbody, None)
            return carry

        lax.fori_loop(0, CH // 16, rgrp, None)
        return _

    lax.fori_loop(0, NCH, chunk_body, None)


@functools.cache
def _make_hist():
    mesh = plsc.VectorSubcoreMesh(core_axis_name="c", subcore_axis_name="s")
    return functools.partial(
        pl.kernel,
        mesh=mesh,
        out_type=jax.ShapeDtypeStruct((KS, B, 128), jnp.float32),
        scratch_types=[
            pltpu.VMEM((CH * L,), jnp.int32),
            pltpu.VMEM((1, CH * VP), jnp.float32),
            pltpu.SemaphoreType.DMA,
        ],
        compiler_params=pltpu.CompilerParams(needs_layout_passes=False),
    )(_hist_body)


BB = 512


def _tc_body(counts_ref, table_ref, w_ref, b_ref, out_ref):
    bow = jnp.dot(counts_ref[0].astype(jnp.bfloat16),
                  table_ref[0].astype(jnp.bfloat16),
                  preferred_element_type=jnp.float32)
    for k in range(1, KS):
        bow += jnp.dot(counts_ref[k].astype(jnp.bfloat16),
                       table_ref[k].astype(jnp.bfloat16),
                       preferred_element_type=jnp.float32)
    sig = 1.0 / (1.0 + jnp.exp(-bow))
    tag = lax.dot_general(sig, w_ref[...], (((1,), (1,)), ((), ())),
                          preferred_element_type=jnp.float32)
    out_ref[...] = tag + b_ref[...]


def _tc_call(counts, table2, w, b2d):
    return pl.pallas_call(
        _tc_body,
        grid=(B // BB,),
        in_specs=[
            pl.BlockSpec((KS, BB, 128), lambda i: (0, i, 0)),
            pl.BlockSpec((KS, 128, D), lambda i: (0, 0, 0)),
            pl.BlockSpec((T, D), lambda i: (0, 0)),
            pl.BlockSpec((1, T), lambda i: (0, 0)),
        ],
        out_specs=pl.BlockSpec((BB, T), lambda i: (i, 0)),
        out_shape=jax.ShapeDtypeStruct((B, T), jnp.float32),
    )(counts, table2, w, b2d)


def kernel(sentence, emb_table, W, b):
    sent_flat = sentence.reshape(B * L).astype(jnp.int32)
    counts = _make_hist()(sent_flat)
    table2 = jnp.pad(emb_table, ((0, VP - V), (0, 0))).reshape(KS, 128, D)
    return _tc_call(counts, table2, W, b.reshape(1, T))
